# baseline (speedup 1.0000x reference)
import functools

import jax
import jax.numpy as jnp
from jax import lax
from jax.experimental import pallas as pl
from jax.experimental.pallas import tpu as pltpu
from jax.experimental.pallas import tpu_sc as plsc

D = 128
NG = 16
T = 2
NP = 10240
CH = 80
BM = 512


def _sc_pass(atab, btab, iidx, jidx):
    E = iidx.shape[0]
    NW = 32
    EPW = E // NW
    NCH = EPW // CH
    BLK = 8
    NSUP = -(-NCH // (2 * BLK))
    RPT = NP // 16

    def body(atab_r, btab_r, iidx_r, jidx_r, acc_out,
             a0, a1, b0, b1, iis, jjs, ib0, ib1, jb0, jb1, acc,
             sa0, sb0, ss0, sa1, sb1, ss1):
        cid = lax.axis_index("c")
        sid = lax.axis_index("s")
        wid = cid * 16 + sid
        base = wid * EPW
        abufs, bbufs = (a0, a1), (b0, b1)
        ibufs, jbufs = (ib0, ib1), (jb0, jb1)
        gsems, ssems = ((sa0, sb0), (sa1, sb1)), (ss0, ss1)

        zv = jnp.zeros((16,), jnp.float32)

        def zrow(r, carry):
            for k in range(D // 16):
                a0[r, pl.ds(k * 16, 16)] = zv
            return carry

        lax.fori_loop(0, CH, zrow, 0)
        r0 = sid * RPT
        for t in range(RPT // CH):
            pltpu.sync_copy(a0, acc.at[pl.ds(r0 + t * CH, CH)])
        plsc.subcore_barrier()

        def load_block(blk, pb):
            o = base + blk * (BLK * CH)
            pltpu.sync_copy(iidx_r.at[pl.ds(o, BLK * CH)], ibufs[pb])
            pltpu.sync_copy(jidx_r.at[pl.ds(o, BLK * CH)], jbufs[pb])

        def copy_idx(u_next, pbuf, p):
            for k in range(CH // 16):
                iis[p, pl.ds(k * 16, 16)] = ibufs[pbuf][pl.ds(u_next * CH + k * 16, 16)]
                jjs[p, pl.ds(k * 16, 16)] = jbufs[pbuf][pl.ds(u_next * CH + k * 16, 16)]

        def issue_gather(p):
            pltpu.async_copy(atab_r.at[iis.at[p]], abufs[p], gsems[p][0])
            pltpu.async_copy(btab_r.at[jjs.at[p]], bbufs[p], gsems[p][1])

        def wait_gather(p):
            pltpu.make_async_copy(atab_r.at[iis.at[p]], abufs[p],
                                  gsems[p][0]).wait()
            pltpu.make_async_copy(btab_r.at[jjs.at[p]], bbufs[p],
                                  gsems[p][1]).wait()

        def wait_scatter(p):
            pltpu.make_async_copy(abufs[p], acc.at[iis.at[p]],
                                  ssems[p]).wait()

        load_block(0, 0)
        copy_idx(0, 0, 0)
        issue_gather(0)

        def sup(j, carry):
            for v in range(2):
                for u in range(BLK):
                    cc = (2 * j + v) * BLK + u
                    p, q = u % 2, 1 - u % 2

                    @pl.when(cc < NCH)
                    def _chunk():
                        wait_gather(p)

                        @pl.when(cc + 1 < NCH)
                        def _next():
                            @pl.when(cc >= 1)
                            def _w():
                                wait_scatter(q)
                            if u < BLK - 1:
                                copy_idx(u + 1, v, q)
                            else:
                                copy_idx(0, 1 - v, q)
                            issue_gather(q)

                        if u == 0:
                            nblk = 2 * j + v + 1

                            @pl.when(nblk * BLK < NCH)
                            def _pref():
                                load_block(nblk, 1 - v)

                        ab, bb = abufs[p], bbufs[p]

                        def rrow(r, carry2):
                            for w in range(8):
                                for k in range(D // 16):
                                    sl = pl.ds(k * 16, 16)
                                    ab[8 * r + w, sl] = jnp.maximum(
                                        ab[8 * r + w, sl] + bb[8 * r + w, sl], 0.0)
                            return carry2

                        lax.fori_loop(0, CH // 8, rrow, 0)
                        pltpu.async_copy(ab, acc.at[iis.at[p]], ssems[p],
                                         add=True)
            return carry

        lax.fori_loop(0, NSUP, sup, 0)
        wait_scatter((NCH - 1) % 2)
        wait_scatter((NCH - 2) % 2)
        plsc.subcore_barrier()
        pltpu.sync_copy(acc.at[pl.ds(r0, RPT)], acc_out.at[cid, pl.ds(r0, RPT)])

    out_type = jax.ShapeDtypeStruct((2, NP, D), jnp.float32)
    scratch = [
        pltpu.VMEM((CH, D), jnp.float32),
        pltpu.VMEM((CH, D), jnp.float32),
        pltpu.VMEM((CH, D), jnp.float32),
        pltpu.VMEM((CH, D), jnp.float32),
        pltpu.VMEM((2, CH), jnp.int32),
        pltpu.VMEM((2, CH), jnp.int32),
        pltpu.VMEM((BLK * CH,), jnp.int32),
        pltpu.VMEM((BLK * CH,), jnp.int32),
        pltpu.VMEM((BLK * CH,), jnp.int32),
        pltpu.VMEM((BLK * CH,), jnp.int32),
        pltpu.VMEM_SHARED((NP, D), jnp.float32),
        pltpu.SemaphoreType.DMA,
        pltpu.SemaphoreType.DMA,
        pltpu.SemaphoreType.DMA,
        pltpu.SemaphoreType.DMA,
        pltpu.SemaphoreType.DMA,
        pltpu.SemaphoreType.DMA,
    ]
    mesh = plsc.VectorSubcoreMesh(core_axis_name="c", subcore_axis_name="s")
    f = pl.kernel(body, out_type=out_type, mesh=mesh,
                  scratch_types=tuple(scratch))
    return f(atab, btab, iidx, jidx)


def _sc_deg(iidx, jidx):
    E = iidx.shape[0]
    EPW = E // 16
    NCH = EPW // CH
    RPT = NP // 16
    DEPTH = 8

    def body(iidx_r, jidx_r, deg_out, iia, ones, zb, dacc, sem):
        cid = lax.axis_index("c")
        sid = lax.axis_index("s")
        base = sid * EPW
        zv = jnp.zeros((16,), jnp.float32)
        ov = jnp.ones((16,), jnp.float32)

        def orow(r, carry):
            for k in range(D // 16):
                ones[r, pl.ds(16 * k, 16)] = ov
                zb[r, pl.ds(16 * k, 16)] = zv
            return carry

        lax.fori_loop(0, CH, orow, 0)
        r0 = sid * RPT
        for t in range(RPT // CH):
            pltpu.sync_copy(zb, dacc.at[pl.ds(r0 + t * CH, CH)])
        plsc.subcore_barrier()

        def step_from(idx_r):
            def step(c, carry):
                @pl.when(c >= DEPTH)
                def _():
                    pltpu.make_async_copy(ones, dacc.at[iia.at[0]],
                                          sem).wait()

                @pl.when(c < NCH)
                def _():
                    slot = c - (c // DEPTH) * DEPTH
                    pltpu.sync_copy(idx_r.at[pl.ds(base + c * CH, CH)],
                                    iia.at[slot])
                    pltpu.async_copy(ones, dacc.at[iia.at[slot]], sem,
                                     add=True)
                return carry
            return step

        @pl.when(cid == 0)
        def _():
            lax.fori_loop(0, NCH + DEPTH, step_from(iidx_r), 0)

        @pl.when(cid == 1)
        def _():
            lax.fori_loop(0, NCH + DEPTH, step_from(jidx_r), 0)

        plsc.subcore_barrier()
        pltpu.sync_copy(dacc.at[pl.ds(r0, RPT)],
                        deg_out.at[cid, pl.ds(r0, RPT)])

    out_type = jax.ShapeDtypeStruct((2, NP, D), jnp.float32)
    scratch = [
        pltpu.VMEM((DEPTH, CH), jnp.int32),
        pltpu.VMEM((CH, D), jnp.float32),
        pltpu.VMEM((CH, D), jnp.float32),
        pltpu.VMEM_SHARED((NP, D), jnp.float32),
        pltpu.SemaphoreType.DMA,
    ]
    mesh = plsc.VectorSubcoreMesh(core_axis_name="c", subcore_axis_name="s")
    f = pl.kernel(body, out_type=out_type, mesh=mesh,
                  scratch_types=tuple(scratch))
    return f(iidx, jidx)


def _tc_encode(nodes_p, W_enc, b_enc, Wp1, bp1, Wc1, bc1):
    def body(n_ref, we, be, wp1, bp1_, wc1, bc1_,
             x_ref, ap_ref, bp_ref, ac_ref, bc_ref):
        x = jnp.dot(n_ref[...], we[...], preferred_element_type=jnp.float32)
        x = x + be[...]
        x_ref[...] = x
        ap_ref[...] = jnp.dot(x, wp1[0:D], preferred_element_type=jnp.float32) + bp1_[...]
        bp_ref[...] = jnp.dot(x, wp1[D:2 * D], preferred_element_type=jnp.float32)
        ac_ref[...] = jnp.dot(x, wc1[0:D], preferred_element_type=jnp.float32) + bc1_[...]
        bc_ref[...] = jnp.dot(x, wc1[D:2 * D], preferred_element_type=jnp.float32)

    grid = (NP // BM,)
    row = pl.BlockSpec((BM, D), lambda i: (i, 0))
    full = lambda a: pl.BlockSpec(a.shape, lambda i: tuple(0 for _ in a.shape))
    outs = [jax.ShapeDtypeStruct((NP, D), jnp.float32)] * 5
    return pl.pallas_call(
        body,
        grid=grid,
        in_specs=[row, full(W_enc), full(b_enc), full(Wp1), full(bp1),
                  full(Wc1), full(bc1)],
        out_specs=[row] * 5,
        out_shape=outs,
    )(nodes_p, W_enc, b_enc, Wp1, bp1, Wc1, bc1)


def _tc_round(x, accP, accC, degP, degC, Wp2, bp2, Wc2, bc2,
              Wf1, bf1, Wf2, bf2, last, Wn1=None, bn1=None, Wn2=None,
              bn2=None, Wconv=None, bconv=None):
    def body(*refs):
        if last:
            (x_ref, hp0, hp1, hc0, hc1, dp_r, dc_r,
             wp2, bp2_, wc2, bc2_, wf1, bf1_, wf2, bf2_, wcv, bcv,
             y_ref) = refs
        else:
            (x_ref, hp0, hp1, hc0, hc1, dp_r, dc_r,
             wp2, bp2_, wc2, bc2_, wf1, bf1_, wf2, bf2_, wn1, bn1_, wn2, bn2_,
             x_out, ap_ref, bp_ref, ac_ref, bc_ref) = refs
        x = x_ref[...]
        hp = hp0[...] + hp1[...]
        hc = hc0[...] + hc1[...]
        dp = dp_r[:, 0:1]
        dc = dc_r[:, 0:1]
        mp_ = jnp.where(dp > 0, 1.0, 0.0)
        mc_ = jnp.where(dc > 0, 1.0, 0.0)
        invp = jnp.where(dp > 0, 1.0 / dp, 0.0)
        invc = jnp.where(dc > 0, 1.0 / dc, 0.0)
        fi = invp * jnp.dot(hp, wp2[...], preferred_element_type=jnp.float32) + mp_ * bp2_[...]
        fo = invc * jnp.dot(hc, wc2[...], preferred_element_type=jnp.float32) + mc_ * bc2_[...]
        h = (jnp.dot(x, wf1[0:D], preferred_element_type=jnp.float32)
             + jnp.dot(fi, wf1[D:2 * D], preferred_element_type=jnp.float32)
             + jnp.dot(fo, wf1[2 * D:3 * D], preferred_element_type=jnp.float32)
             + bf1_[...])
        h = jnp.maximum(h, 0.0)
        xn = x + jnp.dot(h, wf2[...], preferred_element_type=jnp.float32) + bf2_[...]
        if last:
            y_ref[...] = jnp.dot(xn, wcv[...], preferred_element_type=jnp.float32) + bcv[...]
        else:
            x_out[...] = xn
            ap_ref[...] = jnp.dot(xn, wn1[0:D], preferred_element_type=jnp.float32) + bn1_[...]
            bp_ref[...] = jnp.dot(xn, wn1[D:2 * D], preferred_element_type=jnp.float32)
            ac_ref[...] = jnp.dot(xn, wn2[0:D], preferred_element_type=jnp.float32) + bn2_[...]
            bc_ref[...] = jnp.dot(xn, wn2[D:2 * D], preferred_element_type=jnp.float32)

    grid = (NP // BM,)
    row = pl.BlockSpec((BM, D), lambda i: (i, 0))
    full = lambda a: pl.BlockSpec(a.shape, lambda i: tuple(0 for _ in a.shape))
    args = [x, accP[0], accP[1], accC[0], accC[1], degP, degC,
            Wp2, bp2, Wc2, bc2, Wf1, bf1, Wf2, bf2]
    in_specs = [row] * 7 + [full(a) for a in args[7:]]
    if last:
        args += [Wconv, bconv]
        in_specs += [full(Wconv), full(bconv)]
        outs = [jax.ShapeDtypeStruct((NP, 2 * D), jnp.float32)]
        out_specs = [pl.BlockSpec((BM, 2 * D), lambda i: (i, 0))]
    else:
        args += [Wn1, bn1, Wn2, bn2]
        in_specs += [full(Wn1), full(bn1), full(Wn2), full(bn2)]
        outs = [jax.ShapeDtypeStruct((NP, D), jnp.float32)] * 5
        out_specs = [row] * 5
    return pl.pallas_call(
        body, grid=grid, in_specs=in_specs, out_specs=out_specs,
        out_shape=outs,
    )(*args)


def _tc_pool(y, batf):
    nblk = NP // BM

    def body(y_ref, b_ref, out_ref, mx, sm, ct):
        i = pl.program_id(0)

        @pl.when(i == 0)
        def _init():
            mx[...] = jnp.full((NG, 2 * D), -jnp.inf, jnp.float32)
            sm[...] = jnp.zeros((NG, 2 * D), jnp.float32)
            ct[...] = jnp.zeros((NG, 2 * D), jnp.float32)

        yv = y_ref[...]
        b = b_ref[...]
        gids = lax.broadcasted_iota(jnp.int32, (1, NG), 1).astype(jnp.float32)
        ohm = jnp.where(b == gids, 1.0, 0.0)
        sm[...] += lax.dot_general(ohm, yv, (((0,), (0,)), ((), ())),
                                   preferred_element_type=jnp.float32)
        ct[...] += jnp.sum(ohm, axis=0)[:, None]
        rows = []
        for g in range(NG):
            mask = b == float(g)
            rows.append(jnp.max(jnp.where(mask, yv, -jnp.inf), axis=0,
                                keepdims=True))
        mx[...] = jnp.maximum(mx[...], jnp.concatenate(rows, axis=0))

        @pl.when(i == nblk - 1)
        def _fin():
            m = mx[...]
            m = jnp.where(jnp.isfinite(m), m, 0.0)
            mean = sm[...] / jnp.maximum(ct[...], 1.0)
            out_ref[...] = jnp.concatenate([m, mean], axis=1)

    return pl.pallas_call(
        body,
        grid=(nblk,),
        in_specs=[pl.BlockSpec((BM, 2 * D), lambda i: (i, 0)),
                  pl.BlockSpec((BM, 1), lambda i: (i, 0))],
        out_specs=pl.BlockSpec((NG, 4 * D), lambda i: (0, 0)),
        out_shape=jax.ShapeDtypeStruct((NG, 4 * D), jnp.float32),
        scratch_shapes=[pltpu.VMEM((NG, 2 * D), jnp.float32),
                        pltpu.VMEM((NG, 2 * D), jnp.float32),
                        pltpu.VMEM((NG, 2 * D), jnp.float32)],
    )(y, batf)


def kernel(nodes, edges, batch, W_enc, b_enc, Wp1, bp1, Wp2, bp2, Wc1, bc1,
           Wc2, bc2, Wf1, bf1, Wf2, bf2, Wconv, bconv):
    N = nodes.shape[0]
    src = edges[0]
    dst = edges[1]

    nodes_p = jnp.zeros((NP, D), jnp.float32).at[:N].set(nodes)
    batf = jnp.full((NP, 1), 1e9, jnp.float32).at[:N, 0].set(
        batch.astype(jnp.float32))
    r2 = lambda v: v.reshape(1, -1)

    x, ap, bp_t, ac, bc_t = _tc_encode(nodes_p, W_enc, r2(b_enc), Wp1,
                                       r2(bp1), Wc1, r2(bc1))
    deg = _sc_deg(dst, src)
    degP, degC = deg[0], deg[1]
    accP = _sc_pass(ap, bp_t, dst, src)
    accC = _sc_pass(ac, bc_t, src, dst)
    x, ap, bp_t, ac, bc_t = _tc_round(
        x, accP, accC, degP, degC, Wp2, r2(bp2), Wc2, r2(bc2),
        Wf1, r2(bf1), Wf2, r2(bf2), last=False,
        Wn1=Wp1, bn1=r2(bp1), Wn2=Wc1, bn2=r2(bc1))
    accP2 = _sc_pass(ap, bp_t, dst, src)
    accC2 = _sc_pass(ac, bc_t, src, dst)
    y = _tc_round(
        x, accP2, accC2, degP, degC, Wp2, r2(bp2), Wc2, r2(bc2),
        Wf1, r2(bf1), Wf2, r2(bf2), last=True,
        Wconv=Wconv, bconv=r2(bconv))[0]
    return _tc_pool(y, batf)


# speedup vs baseline: 1.0223x; 1.0223x over previous
import functools

import jax
import jax.numpy as jnp
from jax import lax
from jax.experimental import pallas as pl
from jax.experimental.pallas import tpu as pltpu
from jax.experimental.pallas import tpu_sc as plsc

D = 128
NG = 16
T = 2
NP = 10240
CH = 80
BM = 512


def _sc_pass(atab, btab, iidx, jidx):
    E = iidx.shape[0]
    NW = 32
    EPW = E // NW
    NCH = EPW // CH
    BLK = 8
    NSUP = -(-NCH // (2 * BLK))
    RPT = NP // 16

    def body(atab_r, btab_r, iidx_r, jidx_r, acc_out,
             a0, a1, b0, b1, iis, jjs, ib0, ib1, jb0, jb1, acc,
             sa0, sb0, ss0, sa1, sb1, ss1):
        cid = lax.axis_index("c")
        sid = lax.axis_index("s")
        wid = cid * 16 + sid
        base = wid * EPW
        abufs, bbufs = (a0, a1), (b0, b1)
        ibufs, jbufs = (ib0, ib1), (jb0, jb1)
        gsems, ssems = ((sa0, sb0), (sa1, sb1)), (ss0, ss1)

        zv = jnp.zeros((16,), jnp.float32)

        def zrow(r, carry):
            for k in range(D // 16):
                a0[r, pl.ds(k * 16, 16)] = zv
            return carry

        lax.fori_loop(0, CH, zrow, 0)
        r0 = sid * RPT
        for t in range(RPT // CH):
            pltpu.sync_copy(a0, acc.at[pl.ds(r0 + t * CH, CH)])
        plsc.subcore_barrier()

        def load_block(blk, pb):
            o = base + blk * (BLK * CH)
            pltpu.sync_copy(iidx_r.at[pl.ds(o, BLK * CH)], ibufs[pb])
            pltpu.sync_copy(jidx_r.at[pl.ds(o, BLK * CH)], jbufs[pb])

        def copy_idx(u_next, pbuf, p):
            for k in range(CH // 16):
                iis[p, pl.ds(k * 16, 16)] = ibufs[pbuf][pl.ds(u_next * CH + k * 16, 16)]
                jjs[p, pl.ds(k * 16, 16)] = jbufs[pbuf][pl.ds(u_next * CH + k * 16, 16)]

        def issue_gather(p):
            pltpu.async_copy(atab_r.at[iis.at[p]], abufs[p], gsems[p][0])
            pltpu.async_copy(btab_r.at[jjs.at[p]], bbufs[p], gsems[p][1])

        def wait_gather(p):
            pltpu.make_async_copy(atab_r.at[iis.at[p]], abufs[p],
                                  gsems[p][0]).wait()
            pltpu.make_async_copy(btab_r.at[jjs.at[p]], bbufs[p],
                                  gsems[p][1]).wait()

        def wait_scatter(p):
            pltpu.make_async_copy(abufs[p], acc.at[iis.at[p]],
                                  ssems[p]).wait()

        load_block(0, 0)
        copy_idx(0, 0, 0)
        issue_gather(0)

        def sup(j, carry):
            for v in range(2):
                for u in range(BLK):
                    cc = (2 * j + v) * BLK + u
                    p, q = u % 2, 1 - u % 2

                    @pl.when(cc < NCH)
                    def _chunk():
                        wait_gather(p)

                        @pl.when(cc + 1 < NCH)
                        def _next():
                            @pl.when(cc >= 1)
                            def _w():
                                wait_scatter(q)
                            if u < BLK - 1:
                                copy_idx(u + 1, v, q)
                            else:
                                copy_idx(0, 1 - v, q)
                            issue_gather(q)

                        if u == 0:
                            nblk = 2 * j + v + 1

                            @pl.when(nblk * BLK < NCH)
                            def _pref():
                                load_block(nblk, 1 - v)

                        ab, bb = abufs[p], bbufs[p]

                        def rrow(r, carry2):
                            for w in range(4):
                                for k in range(D // 16):
                                    sl = pl.ds(k * 16, 16)
                                    ab[4 * r + w, sl] = jnp.maximum(
                                        ab[4 * r + w, sl] + bb[4 * r + w, sl], 0.0)
                            return carry2

                        lax.fori_loop(0, CH // 4, rrow, 0)
                        pltpu.async_copy(ab, acc.at[iis.at[p]], ssems[p],
                                         add=True)
            return carry

        lax.fori_loop(0, NSUP, sup, 0)
        wait_scatter((NCH - 1) % 2)
        wait_scatter((NCH - 2) % 2)
        plsc.subcore_barrier()
        pltpu.sync_copy(acc.at[pl.ds(r0, RPT)], acc_out.at[cid, pl.ds(r0, RPT)])

    out_type = jax.ShapeDtypeStruct((2, NP, D), jnp.float32)
    scratch = [
        pltpu.VMEM((CH, D), jnp.float32),
        pltpu.VMEM((CH, D), jnp.float32),
        pltpu.VMEM((CH, D), jnp.float32),
        pltpu.VMEM((CH, D), jnp.float32),
        pltpu.VMEM((2, CH), jnp.int32),
        pltpu.VMEM((2, CH), jnp.int32),
        pltpu.VMEM((BLK * CH,), jnp.int32),
        pltpu.VMEM((BLK * CH,), jnp.int32),
        pltpu.VMEM((BLK * CH,), jnp.int32),
        pltpu.VMEM((BLK * CH,), jnp.int32),
        pltpu.VMEM_SHARED((NP, D), jnp.float32),
        pltpu.SemaphoreType.DMA,
        pltpu.SemaphoreType.DMA,
        pltpu.SemaphoreType.DMA,
        pltpu.SemaphoreType.DMA,
        pltpu.SemaphoreType.DMA,
        pltpu.SemaphoreType.DMA,
    ]
    mesh = plsc.VectorSubcoreMesh(core_axis_name="c", subcore_axis_name="s")
    f = pl.kernel(body, out_type=out_type, mesh=mesh,
                  scratch_types=tuple(scratch))
    return f(atab, btab, iidx, jidx)


def _sc_deg(iidx, jidx):
    E = iidx.shape[0]
    EPW = E // 16
    NCH = EPW // CH
    RPT = NP // 16
    DEPTH = 8

    def body(iidx_r, jidx_r, deg_out, iia, ones, zb, dacc, sem):
        cid = lax.axis_index("c")
        sid = lax.axis_index("s")
        base = sid * EPW
        zv = jnp.zeros((16,), jnp.float32)
        ov = jnp.ones((16,), jnp.float32)

        def orow(r, carry):
            for k in range(D // 16):
                ones[r, pl.ds(16 * k, 16)] = ov
                zb[r, pl.ds(16 * k, 16)] = zv
            return carry

        lax.fori_loop(0, CH, orow, 0)
        r0 = sid * RPT
        for t in range(RPT // CH):
            pltpu.sync_copy(zb, dacc.at[pl.ds(r0 + t * CH, CH)])
        plsc.subcore_barrier()

        def step_from(idx_r):
            def step(c, carry):
                @pl.when(c >= DEPTH)
                def _():
                    pltpu.make_async_copy(ones, dacc.at[iia.at[0]],
                                          sem).wait()

                @pl.when(c < NCH)
                def _():
                    slot = c - (c // DEPTH) * DEPTH
                    pltpu.sync_copy(idx_r.at[pl.ds(base + c * CH, CH)],
                                    iia.at[slot])
                    pltpu.async_copy(ones, dacc.at[iia.at[slot]], sem,
                                     add=True)
                return carry
            return step

        @pl.when(cid == 0)
        def _():
            lax.fori_loop(0, NCH + DEPTH, step_from(iidx_r), 0)

        @pl.when(cid == 1)
        def _():
            lax.fori_loop(0, NCH + DEPTH, step_from(jidx_r), 0)

        plsc.subcore_barrier()
        pltpu.sync_copy(dacc.at[pl.ds(r0, RPT)],
                        deg_out.at[cid, pl.ds(r0, RPT)])

    out_type = jax.ShapeDtypeStruct((2, NP, D), jnp.float32)
    scratch = [
        pltpu.VMEM((DEPTH, CH), jnp.int32),
        pltpu.VMEM((CH, D), jnp.float32),
        pltpu.VMEM((CH, D), jnp.float32),
        pltpu.VMEM_SHARED((NP, D), jnp.float32),
        pltpu.SemaphoreType.DMA,
    ]
    mesh = plsc.VectorSubcoreMesh(core_axis_name="c", subcore_axis_name="s")
    f = pl.kernel(body, out_type=out_type, mesh=mesh,
                  scratch_types=tuple(scratch))
    return f(iidx, jidx)


def _tc_encode(nodes_p, W_enc, b_enc, Wp1, bp1, Wc1, bc1):
    def body(n_ref, we, be, wp1, bp1_, wc1, bc1_,
             x_ref, ap_ref, bp_ref, ac_ref, bc_ref):
        x = jnp.dot(n_ref[...], we[...], preferred_element_type=jnp.float32)
        x = x + be[...]
        x_ref[...] = x
        ap_ref[...] = jnp.dot(x, wp1[0:D], preferred_element_type=jnp.float32) + bp1_[...]
        bp_ref[...] = jnp.dot(x, wp1[D:2 * D], preferred_element_type=jnp.float32)
        ac_ref[...] = jnp.dot(x, wc1[0:D], preferred_element_type=jnp.float32) + bc1_[...]
        bc_ref[...] = jnp.dot(x, wc1[D:2 * D], preferred_element_type=jnp.float32)

    grid = (NP // BM,)
    row = pl.BlockSpec((BM, D), lambda i: (i, 0))
    full = lambda a: pl.BlockSpec(a.shape, lambda i: tuple(0 for _ in a.shape))
    outs = [jax.ShapeDtypeStruct((NP, D), jnp.float32)] * 5
    return pl.pallas_call(
        body,
        grid=grid,
        in_specs=[row, full(W_enc), full(b_enc), full(Wp1), full(bp1),
                  full(Wc1), full(bc1)],
        out_specs=[row] * 5,
        out_shape=outs,
    )(nodes_p, W_enc, b_enc, Wp1, bp1, Wc1, bc1)


def _tc_round(x, accP, accC, degP, degC, Wp2, bp2, Wc2, bc2,
              Wf1, bf1, Wf2, bf2, last, Wn1=None, bn1=None, Wn2=None,
              bn2=None, Wconv=None, bconv=None):
    def body(*refs):
        if last:
            (x_ref, hp0, hp1, hc0, hc1, dp_r, dc_r,
             wp2, bp2_, wc2, bc2_, wf1, bf1_, wf2, bf2_, wcv, bcv,
             y_ref) = refs
        else:
            (x_ref, hp0, hp1, hc0, hc1, dp_r, dc_r,
             wp2, bp2_, wc2, bc2_, wf1, bf1_, wf2, bf2_, wn1, bn1_, wn2, bn2_,
             x_out, ap_ref, bp_ref, ac_ref, bc_ref) = refs
        x = x_ref[...]
        hp = hp0[...] + hp1[...]
        hc = hc0[...] + hc1[...]
        dp = dp_r[:, 0:1]
        dc = dc_r[:, 0:1]
        mp_ = jnp.where(dp > 0, 1.0, 0.0)
        mc_ = jnp.where(dc > 0, 1.0, 0.0)
        invp = jnp.where(dp > 0, 1.0 / dp, 0.0)
        invc = jnp.where(dc > 0, 1.0 / dc, 0.0)
        fi = invp * jnp.dot(hp, wp2[...], preferred_element_type=jnp.float32) + mp_ * bp2_[...]
        fo = invc * jnp.dot(hc, wc2[...], preferred_element_type=jnp.float32) + mc_ * bc2_[...]
        h = (jnp.dot(x, wf1[0:D], preferred_element_type=jnp.float32)
             + jnp.dot(fi, wf1[D:2 * D], preferred_element_type=jnp.float32)
             + jnp.dot(fo, wf1[2 * D:3 * D], preferred_element_type=jnp.float32)
             + bf1_[...])
        h = jnp.maximum(h, 0.0)
        xn = x + jnp.dot(h, wf2[...], preferred_element_type=jnp.float32) + bf2_[...]
        if last:
            y_ref[...] = jnp.dot(xn, wcv[...], preferred_element_type=jnp.float32) + bcv[...]
        else:
            x_out[...] = xn
            ap_ref[...] = jnp.dot(xn, wn1[0:D], preferred_element_type=jnp.float32) + bn1_[...]
            bp_ref[...] = jnp.dot(xn, wn1[D:2 * D], preferred_element_type=jnp.float32)
            ac_ref[...] = jnp.dot(xn, wn2[0:D], preferred_element_type=jnp.float32) + bn2_[...]
            bc_ref[...] = jnp.dot(xn, wn2[D:2 * D], preferred_element_type=jnp.float32)

    grid = (NP // BM,)
    row = pl.BlockSpec((BM, D), lambda i: (i, 0))
    full = lambda a: pl.BlockSpec(a.shape, lambda i: tuple(0 for _ in a.shape))
    args = [x, accP[0], accP[1], accC[0], accC[1], degP, degC,
            Wp2, bp2, Wc2, bc2, Wf1, bf1, Wf2, bf2]
    in_specs = [row] * 7 + [full(a) for a in args[7:]]
    if last:
        args += [Wconv, bconv]
        in_specs += [full(Wconv), full(bconv)]
        outs = [jax.ShapeDtypeStruct((NP, 2 * D), jnp.float32)]
        out_specs = [pl.BlockSpec((BM, 2 * D), lambda i: (i, 0))]
    else:
        args += [Wn1, bn1, Wn2, bn2]
        in_specs += [full(Wn1), full(bn1), full(Wn2), full(bn2)]
        outs = [jax.ShapeDtypeStruct((NP, D), jnp.float32)] * 5
        out_specs = [row] * 5
    return pl.pallas_call(
        body, grid=grid, in_specs=in_specs, out_specs=out_specs,
        out_shape=outs,
    )(*args)


def _tc_pool(y, batf):
    nblk = NP // BM

    def body(y_ref, b_ref, out_ref, mx, sm, ct):
        i = pl.program_id(0)

        @pl.when(i == 0)
        def _init():
            mx[...] = jnp.full((NG, 2 * D), -jnp.inf, jnp.float32)
            sm[...] = jnp.zeros((NG, 2 * D), jnp.float32)
            ct[...] = jnp.zeros((NG, 2 * D), jnp.float32)

        yv = y_ref[...]
        b = b_ref[...]
        gids = lax.broadcasted_iota(jnp.int32, (1, NG), 1).astype(jnp.float32)
        ohm = jnp.where(b == gids, 1.0, 0.0)
        sm[...] += lax.dot_general(ohm, yv, (((0,), (0,)), ((), ())),
                                   preferred_element_type=jnp.float32)
        ct[...] += jnp.sum(ohm, axis=0)[:, None]
        rows = []
        for g in range(NG):
            mask = b == float(g)
            rows.append(jnp.max(jnp.where(mask, yv, -jnp.inf), axis=0,
                                keepdims=True))
        mx[...] = jnp.maximum(mx[...], jnp.concatenate(rows, axis=0))

        @pl.when(i == nblk - 1)
        def _fin():
            m = mx[...]
            m = jnp.where(jnp.isfinite(m), m, 0.0)
            mean = sm[...] / jnp.maximum(ct[...], 1.0)
            out_ref[...] = jnp.concatenate([m, mean], axis=1)

    return pl.pallas_call(
        body,
        grid=(nblk,),
        in_specs=[pl.BlockSpec((BM, 2 * D), lambda i: (i, 0)),
                  pl.BlockSpec((BM, 1), lambda i: (i, 0))],
        out_specs=pl.BlockSpec((NG, 4 * D), lambda i: (0, 0)),
        out_shape=jax.ShapeDtypeStruct((NG, 4 * D), jnp.float32),
        scratch_shapes=[pltpu.VMEM((NG, 2 * D), jnp.float32),
                        pltpu.VMEM((NG, 2 * D), jnp.float32),
                        pltpu.VMEM((NG, 2 * D), jnp.float32)],
    )(y, batf)


def kernel(nodes, edges, batch, W_enc, b_enc, Wp1, bp1, Wp2, bp2, Wc1, bc1,
           Wc2, bc2, Wf1, bf1, Wf2, bf2, Wconv, bconv):
    N = nodes.shape[0]
    src = edges[0]
    dst = edges[1]

    nodes_p = jnp.zeros((NP, D), jnp.float32).at[:N].set(nodes)
    batf = jnp.full((NP, 1), 1e9, jnp.float32).at[:N, 0].set(
        batch.astype(jnp.float32))
    r2 = lambda v: v.reshape(1, -1)

    x, ap, bp_t, ac, bc_t = _tc_encode(nodes_p, W_enc, r2(b_enc), Wp1,
                                       r2(bp1), Wc1, r2(bc1))
    deg = _sc_deg(dst, src)
    degP, degC = deg[0], deg[1]
    accP = _sc_pass(ap, bp_t, dst, src)
    accC = _sc_pass(ac, bc_t, src, dst)
    x, ap, bp_t, ac, bc_t = _tc_round(
        x, accP, accC, degP, degC, Wp2, r2(bp2), Wc2, r2(bc2),
        Wf1, r2(bf1), Wf2, r2(bf2), last=False,
        Wn1=Wp1, bn1=r2(bp1), Wn2=Wc1, bn2=r2(bc1))
    accP2 = _sc_pass(ap, bp_t, dst, src)
    accC2 = _sc_pass(ac, bc_t, src, dst)
    y = _tc_round(
        x, accP2, accC2, degP, degC, Wp2, r2(bp2), Wc2, r2(bc2),
        Wf1, r2(bf1), Wf2, r2(bf2), last=True,
        Wconv=Wconv, bconv=r2(bconv))[0]
    return _tc_pool(y, batf)
